# Initial kernel scaffold; baseline (speedup 1.0000x reference)
import jax
import jax.numpy as jnp
from jax.experimental import pallas as pl


def kernel(x, Wg, w1, b1, w2, b2):
    raise NotImplementedError("write your pallas kernel here")


# speedup vs baseline: 2.0911x; 2.0911x over previous
import functools

import jax
import jax.numpy as jnp
from jax import lax
from jax.experimental import pallas as pl
from jax.experimental.pallas import tpu as pltpu
from jax.experimental.pallas import tpu_sc as plsc

B, S, D = 2, 2048, 768
E = 8
DF = 768
T = B * S
TILE = 128
PT = T + E * TILE
NT = PT // TILE
LB_ALPHA = 0.01
Z_ALPHA = 1e-4

NC = 2
NS = 16
NW = NC * NS
TPW = T // NW


def _gate_body(x_ref, wg_ref, pos_ref, te_ref, loss_ref):
    xv = x_ref[...]
    wgv = wg_ref[...]
    logits = lax.dot_general(xv, wgv, (((1,), (1,)), ((), ())),
                             preferred_element_type=jnp.float32)
    m = jnp.max(logits, axis=1, keepdims=True)
    ex = jnp.exp(logits - m)
    se = jnp.sum(ex, axis=1, keepdims=True)
    scores = ex / se
    smax = jnp.max(scores, axis=1, keepdims=True)
    lane = lax.broadcasted_iota(jnp.int32, (T, E), 1)
    top = jnp.min(jnp.where(scores == smax, lane, E), axis=1, keepdims=True)
    ohi = (lane == top).astype(jnp.int32)
    c = ohi
    s = 1
    while s < T:
        c = c + jnp.concatenate(
            [jnp.zeros((s, E), jnp.int32), c[:T - s, :]], axis=0)
        s *= 2
    excl = c - ohi
    counts = c[T - 1:T, :]
    pc = ((counts + TILE - 1) // TILE) * TILE
    tri = (lax.broadcasted_iota(jnp.int32, (E, E), 0)
           < lax.broadcasted_iota(jnp.int32, (E, E), 1)).astype(jnp.float32)
    pad_off = lax.dot_general(pc.astype(jnp.float32), tri,
                              (((1,), (0,)), ((), ())),
                              preferred_element_type=jnp.float32
                              ).astype(jnp.int32)
    pos_ref[...] = jnp.sum(ohi * (pad_off + excl), axis=1, keepdims=True)
    ts = pad_off // TILE
    it = lax.broadcasted_iota(jnp.int32, (NT, E), 0)
    te_ref[...] = jnp.sum((it >= ts).astype(jnp.int32), axis=1,
                          keepdims=True) - 1
    frac = counts.astype(jnp.float32) / T
    prob = jnp.sum(scores, axis=0, keepdims=True) / T
    lb = LB_ALPHA * E * jnp.sum(frac * prob)
    lse = m + jnp.log(se)
    z = Z_ALPHA * jnp.sum(lse * lse) / T
    loss_ref[...] = jnp.broadcast_to(lb + z, (1, 1))


@functools.lru_cache(maxsize=None)
def _sc_kernels():
    mesh = plsc.VectorSubcoreMesh(core_axis_name="c", subcore_axis_name="s",
                                  num_cores=NC, num_subcores=NS)
    scratch = [
        pltpu.VMEM((TPW,), jnp.int32),
        pltpu.VMEM((TPW, D), jnp.float32),
        pltpu.SemaphoreType.DMA,
    ]

    @functools.partial(
        pl.kernel, mesh=mesh,
        out_type=jax.ShapeDtypeStruct((PT, D), jnp.float32),
        scratch_types=scratch,
    )
    def permute_k(x_hbm, pos_hbm, xs_hbm, idx_v, rows_v, sem):
        wid = lax.axis_index("s") * NC + lax.axis_index("c")
        base = wid * TPW
        pltpu.sync_copy(pos_hbm.at[pl.ds(base, TPW)], idx_v)
        pltpu.sync_copy(x_hbm.at[pl.ds(base, TPW)], rows_v)
        pltpu.async_copy(rows_v, xs_hbm.at[idx_v], sem).wait()

    @functools.partial(
        pl.kernel, mesh=mesh,
        out_type=jax.ShapeDtypeStruct((T, D), jnp.float32),
        scratch_types=scratch,
    )
    def unpermute_k(ys_hbm, pos_hbm, out_hbm, idx_v, rows_v, sem):
        wid = lax.axis_index("s") * NC + lax.axis_index("c")
        base = wid * TPW
        pltpu.sync_copy(pos_hbm.at[pl.ds(base, TPW)], idx_v)
        pltpu.async_copy(ys_hbm.at[idx_v], rows_v, sem).wait()
        pltpu.sync_copy(rows_v, out_hbm.at[pl.ds(base, TPW)])

    return permute_k, unpermute_k


def _permute(xf, pos):
    return _sc_kernels()[0](xf, pos)


def _unpermute(ys, pos):
    return _sc_kernels()[1](ys, pos)


def _ffn_body(te_ref, xs_ref, w1_ref, b1_ref, w2_ref, b2_ref, out_ref):
    xv = xs_ref[...]
    h = jnp.dot(xv, w1_ref[0], preferred_element_type=jnp.float32)
    h = jnp.maximum(h + b1_ref[0], 0.0)
    out_ref[...] = (jnp.dot(h, w2_ref[0], preferred_element_type=jnp.float32)
                    + b2_ref[0])


def _ffn(te, xs, w1, b1, w2, b2):
    grid_spec = pltpu.PrefetchScalarGridSpec(
        num_scalar_prefetch=1,
        grid=(NT,),
        in_specs=[
            pl.BlockSpec((TILE, D), lambda i, te: (i, 0)),
            pl.BlockSpec((1, D, DF), lambda i, te: (te[i], 0, 0)),
            pl.BlockSpec((1, 1, DF), lambda i, te: (te[i], 0, 0)),
            pl.BlockSpec((1, DF, D), lambda i, te: (te[i], 0, 0)),
            pl.BlockSpec((1, 1, D), lambda i, te: (te[i], 0, 0)),
        ],
        out_specs=pl.BlockSpec((TILE, D), lambda i, te: (i, 0)),
    )
    return pl.pallas_call(
        _ffn_body,
        grid_spec=grid_spec,
        out_shape=jax.ShapeDtypeStruct((PT, D), jnp.float32),
        compiler_params=pltpu.CompilerParams(
            dimension_semantics=("arbitrary",)),
    )(te, xs, w1, b1.reshape(E, 1, DF), w2, b2.reshape(E, 1, D))


def kernel(x, Wg, w1, b1, w2, b2):
    xf = x.reshape(T, D)
    pos2, te2, loss2 = pl.pallas_call(
        _gate_body,
        out_shape=(
            jax.ShapeDtypeStruct((T, 1), jnp.int32),
            jax.ShapeDtypeStruct((NT, 1), jnp.int32),
            jax.ShapeDtypeStruct((1, 1), jnp.float32),
        ),
    )(xf, Wg)
    pos = pos2.reshape(T)
    te = te2.reshape(NT)
    xs = _permute(xf, pos)
    ys = _ffn(te, xs, w1, b1, w2, b2)
    outf = _unpermute(ys, pos)
    return outf.reshape(B, S, D), loss2[0, 0]


# speedup vs baseline: 2.1774x; 1.0413x over previous
import functools

import jax
import jax.numpy as jnp
from jax import lax
from jax.experimental import pallas as pl
from jax.experimental.pallas import tpu as pltpu
from jax.experimental.pallas import tpu_sc as plsc

B, S, D = 2, 2048, 768
E = 8
DF = 768
T = B * S
TILE = 128
PT = T + E * TILE
NT = PT // TILE
LB_ALPHA = 0.01
Z_ALPHA = 1e-4

NC = 2
NS = 16
NW = NC * NS
TPW = T // NW


CH = 512
NCH = T // CH


def _gate_body(x_ref, wg_ref, pos_ref, meta_ref, loss_ref):
    xv = x_ref[...]
    wgv = wg_ref[...]
    logits = lax.dot_general(xv, wgv, (((1,), (1,)), ((), ())),
                             preferred_element_type=jnp.float32)
    m = jnp.max(logits, axis=1, keepdims=True)
    ex = jnp.exp(logits - m)
    se = jnp.sum(ex, axis=1, keepdims=True)
    scores = ex / se
    smax = jnp.max(scores, axis=1, keepdims=True)
    lane = lax.broadcasted_iota(jnp.int32, (T, E), 1)
    top = jnp.min(jnp.where(scores == smax, lane, E), axis=1, keepdims=True)
    ohi = (lane == top).astype(jnp.int32)
    ohf = ohi.astype(jnp.float32)
    tri_le = (lax.broadcasted_iota(jnp.int32, (CH, CH), 1)
              <= lax.broadcasted_iota(jnp.int32, (CH, CH), 0)
              ).astype(jnp.float32)
    parts = [
        lax.dot_general(tri_le, ohf[k * CH:(k + 1) * CH, :],
                        (((1,), (0,)), ((), ())),
                        preferred_element_type=jnp.float32)
        for k in range(NCH)
    ]
    segs = []
    carry = jnp.zeros((1, E), jnp.float32)
    for k in range(NCH):
        segs.append(parts[k] + carry)
        carry = carry + parts[k][CH - 1:CH, :]
    c = jnp.concatenate(segs, axis=0).astype(jnp.int32)
    excl = c - ohi
    counts = c[T - 1:T, :]
    pc = ((counts + TILE - 1) // TILE) * TILE
    tri = (lax.broadcasted_iota(jnp.int32, (E, E), 0)
           < lax.broadcasted_iota(jnp.int32, (E, E), 1)).astype(jnp.float32)
    pad_off = lax.dot_general(pc.astype(jnp.float32), tri,
                              (((1,), (0,)), ((), ())),
                              preferred_element_type=jnp.float32
                              ).astype(jnp.int32)
    pos_ref[...] = jnp.sum(ohi * (pad_off + excl), axis=1, keepdims=True)
    eyeE = (lax.broadcasted_iota(jnp.int32, (E, E), 0)
            == lax.broadcasted_iota(jnp.int32, (E, E), 1)).astype(jnp.float32)
    counts_col = lax.dot_general(eyeE, counts.astype(jnp.float32),
                                 (((1,), (1,)), ((), ())),
                                 preferred_element_type=jnp.float32)
    pc_col = ((counts_col.astype(jnp.int32) + TILE - 1) // TILE) * TILE
    triL = (lax.broadcasted_iota(jnp.int32, (E, E), 1)
            < lax.broadcasted_iota(jnp.int32, (E, E), 0)).astype(jnp.float32)
    ts_col = (lax.dot_general(triL, pc_col.astype(jnp.float32),
                              (((1,), (0,)), ((), ())),
                              preferred_element_type=jnp.float32)
              .astype(jnp.int32) // TILE)
    itE = lax.broadcasted_iota(jnp.int32, (E, NT), 1)
    eE = lax.broadcasted_iota(jnp.int32, (E, NT), 0)
    ecol = lax.broadcasted_iota(jnp.int32, (E, 1), 0)
    te_row = jnp.sum((itE >= ts_col).astype(jnp.int32), axis=0,
                     keepdims=True) - 1
    pres_col = counts_col > 0.5
    lp = jnp.max(jnp.where(pres_col, ecol, -1), axis=0, keepdims=True)
    te_row = jnp.minimum(te_row, lp)
    nxt_row = jnp.min(jnp.where((eE > te_row) & pres_col, eE, E),
                      axis=0, keepdims=True)
    par_row = jnp.sum((pres_col & (eE <= te_row)).astype(jnp.int32),
                      axis=0, keepdims=True) % 2
    meta_ref[...] = jnp.concatenate(
        [te_row, nxt_row, par_row, jnp.zeros((E - 3, NT), jnp.int32)],
        axis=0)
    frac = counts.astype(jnp.float32) / T
    prob = jnp.sum(scores, axis=0, keepdims=True) / T
    lb = LB_ALPHA * E * jnp.sum(frac * prob)
    lse = m + jnp.log(se)
    z = Z_ALPHA * jnp.sum(lse * lse) / T
    loss_ref[...] = jnp.broadcast_to(lb + z, (1, 1))


@functools.lru_cache(maxsize=None)
def _sc_kernels():
    mesh = plsc.VectorSubcoreMesh(core_axis_name="c", subcore_axis_name="s",
                                  num_cores=NC, num_subcores=NS)
    @functools.partial(
        pl.kernel, mesh=mesh,
        out_type=jax.ShapeDtypeStruct((PT, D), jnp.float32),
        scratch_types=[
            pltpu.VMEM((TPW,), jnp.int32),
            pltpu.VMEM((TPW, D), jnp.float32),
            pltpu.SemaphoreType.DMA,
        ],
    )
    def permute_k(x_hbm, pos_hbm, xs_hbm, idx_v, rows_v, sem):
        wid = lax.axis_index("s") * NC + lax.axis_index("c")
        base = wid * TPW
        pltpu.sync_copy(pos_hbm.at[pl.ds(base, TPW)], idx_v)
        pltpu.sync_copy(x_hbm.at[pl.ds(base, TPW)], rows_v)
        pltpu.async_copy(rows_v, xs_hbm.at[idx_v], sem).wait()

    @functools.partial(
        pl.kernel, mesh=mesh,
        out_type=jax.ShapeDtypeStruct((T, D), jnp.float32),
        scratch_types=[
            pltpu.VMEM((TPW,), jnp.int32),
            pltpu.VMEM((TPW, D), jnp.float32),
            pltpu.SemaphoreType.DMA,
        ],
    )
    def unpermute_k(ys_hbm, pos_hbm, out_hbm, idx_v, rows_v, sem):
        wid = lax.axis_index("s") * NC + lax.axis_index("c")
        base = wid * TPW
        pltpu.sync_copy(pos_hbm.at[pl.ds(base, TPW)], idx_v)
        pltpu.async_copy(ys_hbm.at[idx_v], rows_v, sem).wait()
        pltpu.sync_copy(rows_v, out_hbm.at[pl.ds(base, TPW)])

    return permute_k, unpermute_k


def _permute(xf, pos):
    return _sc_kernels()[0](xf, pos)


def _unpermute(ys, pos):
    return _sc_kernels()[1](ys, pos)


def _ffn_body(meta_ref, xs_ref, w1_hbm, b1_ref, w2_hbm, b2_ref, out_ref,
              w1buf, w2buf, sem1, sem2):
    i = pl.program_id(0)
    e = meta_ref[0, i]
    nxt = meta_ref[1, i]
    slot = meta_ref[2, i]

    def cpy(eidx, s):
        return (pltpu.make_async_copy(w1_hbm.at[eidx], w1buf.at[s],
                                      sem1.at[s]),
                pltpu.make_async_copy(w2_hbm.at[eidx], w2buf.at[s],
                                      sem2.at[s]))

    @pl.when(i == 0)
    def _():
        c1, c2 = cpy(e, slot)
        c1.start()
        c2.start()

    prev = meta_ref[0, jnp.maximum(i - 1, 0)]
    boundary = (i == 0) | (e != prev)

    @pl.when(boundary)
    def _():
        c1, c2 = cpy(e, slot)
        c1.wait()
        c2.wait()

        @pl.when(nxt < E)
        def _():
            n1, n2 = cpy(nxt, 1 - slot)
            n1.start()
            n2.start()

    xv = xs_ref[...]
    h = jnp.dot(xv, w1buf[slot], preferred_element_type=jnp.float32)
    h = jnp.maximum(h + b1_ref[pl.ds(e, 1), :], 0.0)
    out_ref[...] = (jnp.dot(h, w2buf[slot], preferred_element_type=jnp.float32)
                    + b2_ref[pl.ds(e, 1), :])


def _ffn(meta, xs, w1, b1, w2, b2):
    return pl.pallas_call(
        _ffn_body,
        grid=(NT,),
        in_specs=[
            pl.BlockSpec(memory_space=pltpu.SMEM),
            pl.BlockSpec((TILE, D), lambda i: (i, 0)),
            pl.BlockSpec(memory_space=pl.ANY),
            pl.BlockSpec((E, DF), lambda i: (0, 0)),
            pl.BlockSpec(memory_space=pl.ANY),
            pl.BlockSpec((E, D), lambda i: (0, 0)),
        ],
        out_specs=pl.BlockSpec((TILE, D), lambda i: (i, 0)),
        out_shape=jax.ShapeDtypeStruct((PT, D), jnp.float32),
        scratch_shapes=[
            pltpu.VMEM((2, D, DF), jnp.float32),
            pltpu.VMEM((2, DF, D), jnp.float32),
            pltpu.SemaphoreType.DMA((2,)),
            pltpu.SemaphoreType.DMA((2,)),
        ],
        compiler_params=pltpu.CompilerParams(
            dimension_semantics=("arbitrary",)),
    )(meta, xs, w1, b1, w2, b2)


def kernel(x, Wg, w1, b1, w2, b2):
    xf = x.reshape(T, D)
    pos2, meta, loss2 = pl.pallas_call(
        _gate_body,
        out_shape=(
            jax.ShapeDtypeStruct((T, 1), jnp.int32),
            jax.ShapeDtypeStruct((E, NT), jnp.int32),
            jax.ShapeDtypeStruct((1, 1), jnp.float32),
        ),
    )(xf, Wg)
    pos = pos2.reshape(T)
    xs = _permute(xf, pos)
    ys = _ffn(meta, xs, w1, b1, w2, b2)
    outf = _unpermute(ys, pos)
    return outf.reshape(B, S, D), loss2[0, 0]
